# baseline (speedup 1.0000x reference)
import functools

import jax
import jax.numpy as jnp
from jax import lax
from jax.experimental import pallas as pl
from jax.experimental.pallas import tpu as pltpu
from jax.experimental.pallas import tpu_sc as plsc

T_S = 64
CHUNK = 288
DROWS = 112
DVG = DROWS // 16


@functools.lru_cache(maxsize=None)
def _make_kernel(B, D):
    info = plsc.get_sparse_core_info()
    NC, NS = info.num_cores, info.num_subcores
    NW = NC * NS
    b_per_w = B // NW
    s_rows = T_S * CHUNK
    d_rows = T_S * DROWS
    assert b_per_w == s_rows + d_rows and T_S % 8 == 0

    mesh = plsc.VectorSubcoreMesh(core_axis_name="c", subcore_axis_name="s")

    @functools.partial(
        pl.kernel,
        out_type=jax.ShapeDtypeStruct((B, D), jnp.float32),
        mesh=mesh,
        compiler_params=pltpu.CompilerParams(use_tc_tiling_on_sc=False),
        scratch_types=[
            pltpu.VMEM((T_S, CHUNK), jnp.int32),
            pltpu.VMEM((d_rows // 16, 16), jnp.int32),
            pltpu.VMEM((2, CHUNK, D), jnp.float32),
            pltpu.VMEM_SHARED((NS, 4, DROWS, D), jnp.float32),
            pltpu.SemaphoreType.DMA,
            pltpu.SemaphoreType.DMA,
            pltpu.SemaphoreType.DMA,
            pltpu.SemaphoreType.DMA,
            pltpu.SemaphoreType.DMA,
            pltpu.SemaphoreType.DMA,
            pltpu.SemaphoreType.DMA,
            pltpu.SemaphoreType.DMA,
            pltpu.SemaphoreType.DMA,
            pltpu.SemaphoreType.DMA,
            pltpu.SemaphoreType.DMA,
            pltpu.SemaphoreType.DMA,
        ],
    )
    def gather_kernel(
        xs_hbm, xd_hbm, table_hbm, out_hbm,
        idx_vs, idx_vd, rows_v, sp_v,
        gs0, gs1, ss0, ss1, d0, d1, d2, d3, sd0, sd1, sd2, sd3,
    ):
        gsem = (gs0, gs1)
        ssem = (ss0, ss1)
        dsem = (d0, d1, d2, d3)
        sdsem = (sd0, sd1, sd2, sd3)
        wid = lax.axis_index("s") * NC + lax.axis_index("c")
        sid = lax.axis_index("s")
        row0 = wid * b_per_w
        drow0 = row0 + s_rows

        pltpu.sync_copy(xs_hbm.at[wid], idx_vs)
        pltpu.sync_copy(xd_hbm.at[wid], idx_vd)

        def sgather(t, p):
            return pltpu.make_async_copy(
                table_hbm.at[idx_vs.at[t]], rows_v.at[p], gsem[p]
            )

        def sstore(t, p):
            return pltpu.make_async_copy(
                rows_v.at[p], out_hbm.at[pl.ds(row0 + t * CHUNK, CHUNK)],
                ssem[p],
            )

        def dgather_drain(q):
            pltpu.make_async_copy(
                table_hbm.at[pl.ds(0, DROWS)], sp_v.at[sid, q], dsem[q]
            ).wait()

        def dstore(g, q):
            return pltpu.make_async_copy(
                sp_v.at[sid, q],
                out_hbm.at[pl.ds(drow0 + g * DROWS, DROWS)],
                sdsem[q],
            )

        def dissue(g, q):
            def inner(k, _):
                v = idx_vd[g * DVG + k]
                for e in range(16):
                    pltpu.async_copy(
                        table_hbm.at[pl.ds(v[e], 1)],
                        sp_v.at[sid, q, pl.ds(k * 16 + e, 1)],
                        dsem[q],
                    )
                return _

            lax.fori_loop(0, DVG, inner, None)

        sgather(0, 0).start()
        sgather(1, 1).start()

        def body(ii, _):
            for j in range(8):
                t = 8 * ii + j
                p = j % 2
                q = j % 4

                @pl.when(t >= 2)
                def _drain_fire():
                    dgather_drain((q + 2) % 4)
                    dstore(t - 2, (q + 2) % 4).start()

                @pl.when(t >= 4)
                def _free_buf():
                    dstore(t - 4, q).wait()

                dissue(t, q)

                sgather(t, p).wait()
                sstore(t, p).start()
                sstore(t, p).wait()

                @pl.when(t + 2 < T_S)
                def _fire_next():
                    sgather(t + 2, p).start()

            return _

        lax.fori_loop(0, T_S // 8, body, None)

        dgather_drain((T_S - 2) % 4)
        dstore(T_S - 2, (T_S - 2) % 4).start()
        dgather_drain((T_S - 1) % 4)
        dstore(T_S - 1, (T_S - 1) % 4).start()
        for g in range(T_S - 4, T_S):
            dstore(g, g % 4).wait()

    return gather_kernel


def kernel(x, lut_weight):
    B, H = x.shape
    D = lut_weight.shape[1]
    info = plsc.get_sparse_core_info()
    NW = info.num_cores * info.num_subcores
    b_per_w = (B * H) // NW
    s_rows = T_S * CHUNK
    xr = x.astype(jnp.int32).reshape(NW, b_per_w)
    xs = xr[:, :s_rows].reshape(NW, T_S, CHUNK)
    xd = xr[:, s_rows:].reshape(NW, (b_per_w - s_rows) // 16, 16)
    out = _make_kernel(B * H, D)(xs, xd, lut_weight)
    return out.reshape(B, H, D)


# speedup vs baseline: 1.2512x; 1.2512x over previous
import functools

import jax
import jax.numpy as jnp
from jax import lax
from jax.experimental import pallas as pl
from jax.experimental.pallas import tpu as pltpu
from jax.experimental.pallas import tpu_sc as plsc

CHUNK = 512
K = 1
NBUF = 3


@functools.lru_cache(maxsize=None)
def _make_kernel(B, D):
    info = plsc.get_sparse_core_info()
    NC, NS = info.num_cores, info.num_subcores
    NW = NC * NS
    b_per_w = B // NW
    n_chunks = b_per_w // CHUNK
    T = n_chunks // K
    assert B == NW * T * K * CHUNK and T >= 6 and (T - 5) % NBUF == 0

    mesh = plsc.VectorSubcoreMesh(core_axis_name="c", subcore_axis_name="s")

    @functools.partial(
        pl.kernel,
        out_type=jax.ShapeDtypeStruct((B, D), jnp.float32),
        mesh=mesh,
        compiler_params=pltpu.CompilerParams(use_tc_tiling_on_sc=False),
        scratch_types=[
            pltpu.VMEM((n_chunks, CHUNK), jnp.int32),
            pltpu.VMEM((NBUF, K * CHUNK, D), jnp.float32),
            pltpu.SemaphoreType.DMA,
            pltpu.SemaphoreType.DMA,
            pltpu.SemaphoreType.DMA,
            pltpu.SemaphoreType.DMA,
            pltpu.SemaphoreType.DMA,
            pltpu.SemaphoreType.DMA,
        ],
    )
    def gather_kernel(
        x_hbm, table_hbm, out_hbm, idx_v, rows_v, g0, g1, g2, s0, s1, s2
    ):
        gsem = (g0, g1, g2)
        ssem = (s0, s1, s2)
        wid = lax.axis_index("s") * NC + lax.axis_index("c")
        row0 = wid * b_per_w

        pltpu.sync_copy(x_hbm.at[wid], idx_v)

        def gathers(t, p):
            return [
                pltpu.make_async_copy(
                    table_hbm.at[idx_v.at[t * K + j]],
                    rows_v.at[p, pl.ds(j * CHUNK, CHUNK)],
                    gsem[p],
                )
                for j in range(K)
            ]

        def stores(t, p):
            return [
                pltpu.make_async_copy(
                    rows_v.at[p],
                    out_hbm.at[pl.ds(row0 + t * K * CHUNK, K * CHUNK)],
                    ssem[p],
                )
            ]

        def step(u, p, drain_prev=True, fire_next=True):
            for d in gathers(u, p):
                d.wait()
            for d in stores(u, p):
                d.start()
            if drain_prev:
                for d in stores(u - 1, (p + 2) % NBUF):
                    d.wait()
            if fire_next:
                for d in gathers(u + 2, (p + 2) % NBUF):
                    d.start()

        for t in range(2):
            for d in gathers(t, t):
                d.start()
        step(0, 0, drain_prev=False)

        def body(i, _):
            u0 = NBUF * i + 1
            for dp in range(NBUF):
                step(u0 + dp, (1 + dp) % NBUF)
            return _

        lax.fori_loop(0, (T - 5) // NBUF, body, None)

        for u in range(T - 4, T):
            step(u, u % NBUF, fire_next=(u + 2 < T))
        for d in stores(T - 1, (T - 1) % NBUF):
            d.wait()

    return gather_kernel


def kernel(x, lut_weight):
    B, H = x.shape
    D = lut_weight.shape[1]
    info = plsc.get_sparse_core_info()
    NW = info.num_cores * info.num_subcores
    n_chunks = (B * H) // (NW * CHUNK)
    idx = x.astype(jnp.int32).reshape(NW, n_chunks, CHUNK)
    out = _make_kernel(B * H, D)(idx, lut_weight)
    return out.reshape(B, H, D)
